# baseline (speedup 1.0000x reference)
import dataclasses
import functools

import jax
import jax.numpy as jnp
from jax import lax
from jax.experimental import pallas as pl
from jax.experimental.pallas import tpu as pltpu
from jax.experimental.pallas import tpu_sc as plsc

ROWS = 8192
COLS = 4096
NUM_CORES = 2
NUM_SUBCORES = 16
NUM_WORKERS = NUM_CORES * NUM_SUBCORES
ROWS_PER_WORKER = ROWS // NUM_WORKERS
CHUNK = 4
NBUF = 2
NCHUNKS = ROWS_PER_WORKER // CHUNK
LANES = 16


def _compiler_params():
    cp = pltpu.CompilerParams()
    if "needs_layout_passes" in pltpu.CompilerParams.__dataclass_fields__:
        cp = dataclasses.replace(cp, needs_layout_passes=False)
    return cp


def _permute_sc(x, perm):
    mesh = plsc.VectorSubcoreMesh(
        core_axis_name="c", subcore_axis_name="s")

    @functools.partial(
        pl.kernel,
        compiler_params=_compiler_params(),
        out_type=jax.ShapeDtypeStruct((ROWS, COLS), jnp.float32),
        mesh=mesh,
        scratch_types=[
            pltpu.VMEM((COLS,), jnp.int32),
            pltpu.VMEM((NBUF, CHUNK, COLS), jnp.float32),
            pltpu.VMEM((NBUF, CHUNK, COLS), jnp.float32),
            pltpu.SemaphoreType.DMA((NBUF,)),
            pltpu.SemaphoreType.DMA((NBUF,)),
        ],
    )
    def run(x_hbm, perm_hbm, out_hbm, idx_v, in_b, out_b, in_sems, out_sems):
        wid = lax.axis_index("s") * NUM_CORES + lax.axis_index("c")
        base = wid * ROWS_PER_WORKER
        pltpu.sync_copy(perm_hbm, idx_v)

        def in_copy(c, b):
            return pltpu.make_async_copy(
                x_hbm.at[pl.ds(base + c * CHUNK, CHUNK)],
                in_b.at[b], in_sems.at[b])

        def out_copy(c, b):
            return pltpu.make_async_copy(
                out_b.at[b],
                out_hbm.at[pl.ds(base + c * CHUNK, CHUNK)], out_sems.at[b])

        for b in range(NBUF):
            in_copy(b, b).start()

        @pl.loop(0, NCHUNKS, step=NBUF)
        def _group(c0):
            for b in range(NBUF):
                c = c0 + b
                in_copy(c, b).wait()

                @pl.when(c0 > 0)
                def _wait_out():
                    out_copy(c - NBUF, b).wait()

                @plsc.parallel_loop(0, COLS, LANES, unroll=8)
                def _col(j):
                    idx = idx_v[pl.ds(j, LANES)]
                    for r in range(CHUNK):
                        row = jnp.full((LANES,), r, jnp.int32)
                        g = plsc.load_gather(in_b.at[b], [row, idx])
                        out_b[b, r, pl.ds(j, LANES)] = g

                out_copy(c, b).start()

                @pl.when(c + NBUF < NCHUNKS)
                def _next_in():
                    in_copy(c + NBUF, b).start()

        for b in range(NBUF):
            out_copy(NCHUNKS - NBUF + b, b).wait()

    return run(x, perm)


def kernel(x, perm_tensor, perm_inv_tensor):
    del perm_inv_tensor
    perm = perm_tensor.astype(jnp.int32)
    return _permute_sc(x, perm)


# speedup vs baseline: 1.0267x; 1.0267x over previous
import dataclasses
import functools

import jax
import jax.numpy as jnp
from jax import lax
from jax.experimental import pallas as pl
from jax.experimental.pallas import tpu as pltpu
from jax.experimental.pallas import tpu_sc as plsc

ROWS = 8192
COLS = 4096
NUM_CORES = 2
NUM_SUBCORES = 16
NUM_WORKERS = NUM_CORES * NUM_SUBCORES
ROWS_PER_WORKER = ROWS // NUM_WORKERS
CHUNK = 2
NBUF = 4
NCHUNKS = ROWS_PER_WORKER // CHUNK
LANES = 16


def _compiler_params():
    cp = pltpu.CompilerParams()
    if "needs_layout_passes" in pltpu.CompilerParams.__dataclass_fields__:
        cp = dataclasses.replace(cp, needs_layout_passes=False)
    return cp


def _permute_sc(x, perm):
    mesh = plsc.VectorSubcoreMesh(
        core_axis_name="c", subcore_axis_name="s")

    @functools.partial(
        pl.kernel,
        compiler_params=_compiler_params(),
        out_type=jax.ShapeDtypeStruct((ROWS, COLS), jnp.float32),
        mesh=mesh,
        scratch_types=[
            pltpu.VMEM((COLS,), jnp.int32),
            pltpu.VMEM((NBUF, CHUNK, COLS), jnp.float32),
            pltpu.VMEM((NBUF, CHUNK, COLS), jnp.float32),
            pltpu.SemaphoreType.DMA((NBUF,)),
            pltpu.SemaphoreType.DMA((NBUF,)),
        ],
    )
    def run(x_hbm, perm_hbm, out_hbm, idx_v, in_b, out_b, in_sems, out_sems):
        wid = lax.axis_index("s") * NUM_CORES + lax.axis_index("c")
        base = wid * ROWS_PER_WORKER
        pltpu.sync_copy(perm_hbm, idx_v)

        def in_copy(c, b):
            return pltpu.make_async_copy(
                x_hbm.at[pl.ds(base + c * CHUNK, CHUNK)],
                in_b.at[b], in_sems.at[b])

        def out_copy(c, b):
            return pltpu.make_async_copy(
                out_b.at[b],
                out_hbm.at[pl.ds(base + c * CHUNK, CHUNK)], out_sems.at[b])

        for b in range(NBUF):
            in_copy(b, b).start()

        @pl.loop(0, NCHUNKS, step=NBUF)
        def _group(c0):
            for b in range(NBUF):
                c = c0 + b
                in_copy(c, b).wait()

                @pl.when(c0 > 0)
                def _wait_out():
                    out_copy(c - NBUF, b).wait()

                @plsc.parallel_loop(0, COLS, LANES, unroll=16)
                def _col(j):
                    idx = idx_v[pl.ds(j, LANES)]
                    for r in range(CHUNK):
                        row = jnp.full((LANES,), r, jnp.int32)
                        g = plsc.load_gather(in_b.at[b], [row, idx])
                        out_b[b, r, pl.ds(j, LANES)] = g

                out_copy(c, b).start()

                @pl.when(c + NBUF < NCHUNKS)
                def _next_in():
                    in_copy(c + NBUF, b).start()

        for b in range(NBUF):
            out_copy(NCHUNKS - NBUF + b, b).wait()

    return run(x, perm)


def kernel(x, perm_tensor, perm_inv_tensor):
    del perm_inv_tensor
    perm = perm_tensor.astype(jnp.int32)
    return _permute_sc(x, perm)


# speedup vs baseline: 1.0303x; 1.0035x over previous
import dataclasses
import functools

import jax
import jax.numpy as jnp
from jax import lax
from jax.experimental import pallas as pl
from jax.experimental.pallas import tpu as pltpu
from jax.experimental.pallas import tpu_sc as plsc

ROWS = 8192
COLS = 4096
NUM_CORES = 2
NUM_SUBCORES = 16
NUM_WORKERS = NUM_CORES * NUM_SUBCORES
ROWS_PER_WORKER = ROWS // NUM_WORKERS
CHUNK = 2
NBUF = 4
NCHUNKS = ROWS_PER_WORKER // CHUNK
LANES = 16


def _compiler_params():
    cp = pltpu.CompilerParams()
    if "needs_layout_passes" in pltpu.CompilerParams.__dataclass_fields__:
        cp = dataclasses.replace(cp, needs_layout_passes=False)
    return cp


def _permute_sc(x, perm):
    mesh = plsc.VectorSubcoreMesh(
        core_axis_name="c", subcore_axis_name="s")

    @functools.partial(
        pl.kernel,
        compiler_params=_compiler_params(),
        out_type=jax.ShapeDtypeStruct((ROWS, COLS), jnp.float32),
        mesh=mesh,
        scratch_types=[
            pltpu.VMEM((COLS,), jnp.int32),
            pltpu.VMEM((NBUF, CHUNK, COLS), jnp.float32),
            pltpu.VMEM((NBUF, CHUNK, COLS), jnp.float32),
            pltpu.SemaphoreType.DMA((NBUF,)),
            pltpu.SemaphoreType.DMA((NBUF,)),
        ],
    )
    def run(x_hbm, perm_hbm, out_hbm, idx_v, in_b, out_b, in_sems, out_sems):
        wid = lax.axis_index("s") * NUM_CORES + lax.axis_index("c")
        base = wid * ROWS_PER_WORKER
        pltpu.sync_copy(perm_hbm, idx_v)

        def in_copy(c, b):
            return pltpu.make_async_copy(
                x_hbm.at[pl.ds(base + c * CHUNK, CHUNK)],
                in_b.at[b], in_sems.at[b])

        def out_copy(c, b):
            return pltpu.make_async_copy(
                out_b.at[b],
                out_hbm.at[pl.ds(base + c * CHUNK, CHUNK)], out_sems.at[b])

        for b in range(NBUF):
            in_copy(b, b).start()

        @pl.loop(0, NCHUNKS, step=NBUF)
        def _group(c0):
            for b in range(NBUF):
                c = c0 + b
                in_copy(c, b).wait()

                @pl.when(c0 > 0)
                def _wait_out():
                    out_copy(c - NBUF, b).wait()

                @plsc.parallel_loop(0, COLS, LANES, unroll=8)
                def _col(j):
                    idx = idx_v[pl.ds(j, LANES)]
                    for r in range(CHUNK):
                        row = jnp.full((LANES,), r, jnp.int32)
                        g = plsc.load_gather(in_b.at[b], [row, idx])
                        out_b[b, r, pl.ds(j, LANES)] = g

                out_copy(c, b).start()

                @pl.when(c + NBUF < NCHUNKS)
                def _next_in():
                    in_copy(c + NBUF, b).start()

        for b in range(NBUF):
            out_copy(NCHUNKS - NBUF + b, b).wait()

    return run(x, perm)


def kernel(x, perm_tensor, perm_inv_tensor):
    del perm_inv_tensor
    perm = perm_tensor.astype(jnp.int32)
    return _permute_sc(x, perm)
